# Initial kernel scaffold; baseline (speedup 1.0000x reference)
import jax
import jax.numpy as jnp
from jax.experimental import pallas as pl


def kernel(node_feats, edge_index, edge_types, params):
    raise NotImplementedError("write your pallas kernel here")


# speedup vs baseline: 2.3745x; 2.3745x over previous
import functools

import jax
import jax.numpy as jnp
from jax import lax
from jax.experimental import pallas as pl
from jax.experimental.pallas import tpu as pltpu
from jax.experimental.pallas import tpu_sc as plsc

N = 50000
E = 800000
R = 4
IN_DIM = 4
HID = 64
HID2 = 128
LATENT = 32

HALF = N // 2
ROWS = HALF * R
NDUM = 96
CAP = ROWS + NDUM
B = 128
NB = E // B
ZB = 391
DR = 625


def _make_edge_pass(fp: int):
    mesh = plsc.VectorSubcoreMesh(core_axis_name="c", subcore_axis_name="s")

    @functools.partial(
        pl.kernel,
        mesh=mesh,
        compiler_params=pltpu.CompilerParams(use_tc_tiling_on_sc=False),
        out_type=jax.ShapeDtypeStruct((fp, 2, ROWS, 16), jnp.float32),
        scratch_types=[
            pltpu.VMEM_SHARED((CAP, 16), jnp.float32),
            pltpu.VMEM((ZB, 16), jnp.float32),
            pltpu.VMEM((DR, 16), jnp.float32),
            pltpu.VMEM((B,), jnp.int32),
            pltpu.VMEM((B,), jnp.int32),
            pltpu.VMEM((B,), jnp.int32),
            pltpu.VMEM((B,), jnp.int32),
            pltpu.VMEM((B,), jnp.int32),
            pltpu.VMEM((B, 16), jnp.float32),
            pltpu.SemaphoreType.DMA,
        ],
    )
    def edge_pass(table, src, dst, et, out,
                  acc, zbuf, dbuf, gidx, sidx, sbuf, dbuf_i, ebuf, grows, sem):
        c = lax.axis_index("c")
        s = lax.axis_index("s")
        lo = c * HALF
        lo4 = lo * R

        def zrow(i, _):
            zbuf[i, :] = jnp.zeros((16,), jnp.float32)
            return 0
        lax.fori_loop(0, ZB, zrow, 0)

        nbase, nrem = NB // 16, NB % 16
        nb = nbase + jnp.where(s < nrem, 1, 0)
        start = s * nbase + jnp.minimum(s, nrem)

        for p in range(fp):
            for k in range(16):
                pltpu.sync_copy(zbuf, acc.at[pl.ds(s * 6256 + k * ZB, ZB)])
            plsc.subcore_barrier()

            def body(i, _):
                b0 = (start + i) * B
                pltpu.sync_copy(src.at[pl.ds(b0, B)], sbuf)
                pltpu.sync_copy(dst.at[pl.ds(b0, B)], dbuf_i)
                pltpu.sync_copy(et.at[pl.ds(b0, B)], ebuf)
                for v in range(B // 16):
                    sl = pl.ds(v * 16, 16)
                    dv = dbuf_i[sl]
                    ev = ebuf[sl]
                    sv = sbuf[sl]
                    hit = (dv >= lo) & (dv < lo + HALF)
                    offv = dv * R + ev - lo4
                    dum = (ROWS
                           + ((i * (B // 16) + v) % 6) * 16
                           + lax.iota(jnp.int32, 16))
                    sidx[sl] = jnp.where(hit, offv, dum)
                    gidx[sl] = sv * fp + p
                pltpu.async_copy(table.at[gidx], grows, sem).wait()
                pltpu.sync_copy(grows, acc.at[sidx], add=True)
                return 0
            lax.fori_loop(0, nb, body, 0)
            plsc.subcore_barrier()

            for k in range(10):
                r0 = s * 6250 + k * DR
                pltpu.sync_copy(acc.at[pl.ds(r0, DR)], dbuf)
                pltpu.sync_copy(dbuf, out.at[p, c, pl.ds(r0, DR)])
            plsc.subcore_barrier()

    return edge_pass


_edge_pass_1 = _make_edge_pass(1)
_edge_pass_4 = _make_edge_pass(4)


def _leaky_relu(x):
    return jnp.where(x >= 0, x, 0.1 * x)


def _layer_norm(x, g, b, eps=1e-5):
    m = jnp.mean(x, axis=-1, keepdims=True)
    v = jnp.mean((x - m) ** 2, axis=-1, keepdims=True)
    return (x - m) / jnp.sqrt(v + eps) * g + b


TB = 2000


def _layer1_body(x_ref, s1_ref, wc_ref, bp_ref, ws_ref, b_ref, g_ref,
                 bb_ref, h1_ref):
    agg = jnp.zeros((TB, HID), jnp.float32)
    for r in range(R):
        w_r = jnp.zeros((16, HID), jnp.float32)
        for bi in range(R):
            w_r = w_r + wc_ref[r, bi] * bp_ref[bi]
        agg = agg + jnp.dot(s1_ref[:, r * 16:(r + 1) * 16], w_r,
                            preferred_element_type=jnp.float32, precision=lax.Precision.HIGHEST)
    h = agg + jnp.dot(x_ref[...], ws_ref[...],
                      preferred_element_type=jnp.float32, precision=lax.Precision.HIGHEST) + b_ref[...]
    h = _leaky_relu(h)
    h1_ref[...] = _layer_norm(h, g_ref[...], bb_ref[...])


def _layer2_body(h1_ref, s2_ref, wc_ref, b2_ref, ws_ref, b_ref, g_ref,
                 bb_ref, gw1_ref, gb1_ref, gw2_ref, gb2_ref,
                 h2_ref, gate_ref):
    agg = jnp.zeros((TB, HID2), jnp.float32)
    for r in range(R):
        w_r = jnp.zeros((HID, HID2), jnp.float32)
        for bi in range(R):
            w_r = w_r + wc_ref[r, bi] * b2_ref[bi]
        agg = agg + jnp.dot(s2_ref[:, r * HID:(r + 1) * HID], w_r,
                            preferred_element_type=jnp.float32, precision=lax.Precision.HIGHEST)
    h = agg + jnp.dot(h1_ref[...], ws_ref[...],
                      preferred_element_type=jnp.float32, precision=lax.Precision.HIGHEST) + b_ref[...]
    h = _leaky_relu(h)
    h2 = _layer_norm(h, g_ref[...], bb_ref[...])
    h2_ref[...] = h2
    ga = jnp.maximum(jnp.dot(h2, gw1_ref[...],
                             preferred_element_type=jnp.float32, precision=lax.Precision.HIGHEST)
                     + gb1_ref[...], 0.0)
    gate_ref[...] = jnp.dot(ga, gw2_ref[...],
                            preferred_element_type=jnp.float32, precision=lax.Precision.HIGHEST) + gb2_ref[...]


def _pool_body(gate_ref, h2_ref, wmu_ref, bmu_ref, wlv_ref, blv_ref,
               mu_ref, lv_ref, acc_ref):
    i = pl.program_id(0)
    g = gate_ref[:, 0:1]
    m = jnp.max(g)
    ssum = jnp.sum(jnp.exp(g - m))
    gblk = gate_ref[pl.ds(i * TB, TB), :]
    w = jnp.exp(gblk - m) / ssum

    @pl.when(i == 0)
    def _():
        acc_ref[...] = jnp.zeros((8, HID2), jnp.float32)

    acc_ref[...] += lax.dot_general(
        w, h2_ref[...], (((0,), (0,)), ((), ())),
        preferred_element_type=jnp.float32, precision=lax.Precision.HIGHEST)
    ge = acc_ref[0:1, :]
    mu_ref[...] = jnp.clip(
        jnp.dot(ge, wmu_ref[...], preferred_element_type=jnp.float32, precision=lax.Precision.HIGHEST)
        + bmu_ref[...], -5.0, 5.0)
    lv_ref[...] = jnp.clip(
        jnp.dot(ge, wlv_ref[...], preferred_element_type=jnp.float32, precision=lax.Precision.HIGHEST)
        + blv_ref[...], -10.0, 10.0)


def kernel(node_feats, edge_index, edge_types, params):
    src = edge_index[0]
    dst = edge_index[1]
    et = edge_types

    xpad = jnp.pad(node_feats, ((0, 0), (0, 16 - IN_DIM)))

    s1_raw = _edge_pass_1(xpad, src, dst, et)
    s1 = s1_raw.reshape(N, R * 16)

    bases1p = jnp.pad(params['bases1'], ((0, 0), (0, 16 - IN_DIM), (0, 0)))

    grid1 = N // TB
    h1 = pl.pallas_call(
        _layer1_body,
        grid=(grid1,),
        in_specs=[
            pl.BlockSpec((TB, IN_DIM), lambda i: (i, 0)),
            pl.BlockSpec((TB, R * 16), lambda i: (i, 0)),
            pl.BlockSpec((R, R), lambda i: (0, 0)),
            pl.BlockSpec((R, 16, HID), lambda i: (0, 0, 0)),
            pl.BlockSpec((IN_DIM, HID), lambda i: (0, 0)),
            pl.BlockSpec((HID,), lambda i: (0,)),
            pl.BlockSpec((HID,), lambda i: (0,)),
            pl.BlockSpec((HID,), lambda i: (0,)),
        ],
        out_specs=pl.BlockSpec((TB, HID), lambda i: (i, 0)),
        out_shape=jax.ShapeDtypeStruct((N, HID), jnp.float32),
    )(node_feats, s1, params['w_comp1'], bases1p, params['w_self1'],
      params['b1'], params['ln1_g'], params['ln1_b'])

    h1t = h1.reshape(N * R, 16)
    s2_raw = _edge_pass_4(h1t, src, dst, et)
    s2 = s2_raw.transpose(1, 2, 0, 3).reshape(N, R * HID)

    gw2_8 = jnp.tile(params['gate_w2'], (1, 8))
    gb2_8 = jnp.tile(params['gate_b2'], (8,))

    h2, gate8 = pl.pallas_call(
        _layer2_body,
        grid=(grid1,),
        in_specs=[
            pl.BlockSpec((TB, HID), lambda i: (i, 0)),
            pl.BlockSpec((TB, R * HID), lambda i: (i, 0)),
            pl.BlockSpec((R, R), lambda i: (0, 0)),
            pl.BlockSpec((R, HID, HID2), lambda i: (0, 0, 0)),
            pl.BlockSpec((HID, HID2), lambda i: (0, 0)),
            pl.BlockSpec((HID2,), lambda i: (0,)),
            pl.BlockSpec((HID2,), lambda i: (0,)),
            pl.BlockSpec((HID2,), lambda i: (0,)),
            pl.BlockSpec((HID2, HID), lambda i: (0, 0)),
            pl.BlockSpec((HID,), lambda i: (0,)),
            pl.BlockSpec((HID, 8), lambda i: (0, 0)),
            pl.BlockSpec((8,), lambda i: (0,)),
        ],
        out_specs=[
            pl.BlockSpec((TB, HID2), lambda i: (i, 0)),
            pl.BlockSpec((TB, 8), lambda i: (i, 0)),
        ],
        out_shape=[
            jax.ShapeDtypeStruct((N, HID2), jnp.float32),
            jax.ShapeDtypeStruct((N, 8), jnp.float32),
        ],
    )(h1, s2, params['w_comp2'], params['bases2'], params['w_self2'],
      params['b2'], params['ln2_g'], params['ln2_b'],
      params['gate_w1'], params['gate_b1'], gw2_8, gb2_8)

    mu, logvar = pl.pallas_call(
        _pool_body,
        grid=(grid1,),
        in_specs=[
            pl.BlockSpec((N, 8), lambda i: (0, 0)),
            pl.BlockSpec((TB, HID2), lambda i: (i, 0)),
            pl.BlockSpec((HID2, LATENT), lambda i: (0, 0)),
            pl.BlockSpec((LATENT,), lambda i: (0,)),
            pl.BlockSpec((HID2, LATENT), lambda i: (0, 0)),
            pl.BlockSpec((LATENT,), lambda i: (0,)),
        ],
        out_specs=[
            pl.BlockSpec((1, LATENT), lambda i: (0, 0)),
            pl.BlockSpec((1, LATENT), lambda i: (0, 0)),
        ],
        out_shape=[
            jax.ShapeDtypeStruct((1, LATENT), jnp.float32),
            jax.ShapeDtypeStruct((1, LATENT), jnp.float32),
        ],
        scratch_shapes=[pltpu.VMEM((8, HID2), jnp.float32)],
    )(gate8, h2, params['w_mu'], params['b_mu'],
      params['w_logvar'], params['b_logvar'])

    return (mu, logvar, h2)


# speedup vs baseline: 4.0712x; 1.7145x over previous
import functools

import jax
import jax.numpy as jnp
from jax import lax
from jax.experimental import pallas as pl
from jax.experimental.pallas import tpu as pltpu
from jax.experimental.pallas import tpu_sc as plsc

N = 50000
E = 800000
R = 4
IN_DIM = 4
HID = 64
HID2 = 128
LATENT = 32

HALF = N // 2
ROWS = HALF * R
NDUM = 96
CAP = ROWS + NDUM
B = 128
SB = 1024
NSB = -(-E // SB)
EPAD = NSB * SB - E
ZB = 391
DR = 625


def _make_edge_pass(fp: int):
    mesh = plsc.VectorSubcoreMesh(core_axis_name="c", subcore_axis_name="s")

    @functools.partial(
        pl.kernel,
        mesh=mesh,
        compiler_params=pltpu.CompilerParams(use_tc_tiling_on_sc=False),
        out_type=jax.ShapeDtypeStruct((fp, 2, ROWS, 16), jnp.float32),
        scratch_types=[
            pltpu.VMEM_SHARED((CAP, 16), jnp.float32),
            pltpu.VMEM((ZB, 16), jnp.float32),
            pltpu.VMEM((DR, 16), jnp.float32),
            pltpu.VMEM((B,), jnp.int32),
            pltpu.VMEM((B,), jnp.int32),
            pltpu.VMEM((B,), jnp.int32),
            pltpu.VMEM((B,), jnp.int32),
            pltpu.VMEM((SB,), jnp.int32),
            pltpu.VMEM((SB,), jnp.int32),
            pltpu.VMEM((SB,), jnp.int32),
            pltpu.VMEM((B, 16), jnp.float32),
            pltpu.VMEM((B, 16), jnp.float32),
            pltpu.SemaphoreType.DMA,
            pltpu.SemaphoreType.DMA,
        ],
    )
    def edge_pass(table, src, dst, et, out,
                  acc, zbuf, dbuf, gidx0, gidx1, sidx0, sidx1,
                  sbuf, dbuf_i, ebuf, grows0, grows1, gsem, ssem):
        c = lax.axis_index("c")
        s = lax.axis_index("s")
        lo = c * HALF
        lo4 = lo * R

        def zrow(i, _):
            zbuf[i, :] = jnp.zeros((16,), jnp.float32)
            return 0
        lax.fori_loop(0, ZB, zrow, 0)

        nbase, nrem = NSB // 16, NSB % 16
        nsb = nbase + jnp.where(s < nrem, 1, 0)
        sb_start = s * nbase + jnp.minimum(s, nrem)

        def pass_body(p, _):
            for k in range(16):
                pltpu.sync_copy(zbuf, acc.at[pl.ds(s * 6256 + k * ZB, ZB)])
            plsc.subcore_barrier()

            def sb_body(k, _):
                sb0 = (sb_start + k) * SB
                pltpu.sync_copy(src.at[pl.ds(sb0, SB)], sbuf)
                pltpu.sync_copy(dst.at[pl.ds(sb0, SB)], dbuf_i)
                pltpu.sync_copy(et.at[pl.ds(sb0, SB)], ebuf)

                def compute_idx(j):
                    gi = (gidx0, gidx1)[j % 2]
                    si = (sidx0, sidx1)[j % 2]
                    for v in range(B // 16):
                        sl = pl.ds(j * B + v * 16, 16)
                        osl = pl.ds(v * 16, 16)
                        dv = dbuf_i[sl]
                        ev = ebuf[sl]
                        sv = sbuf[sl]
                        hit = (dv >= lo) & (dv < lo + HALF)
                        offv = dv * R + ev - lo4
                        dum = (ROWS + ((j * 8 + v) % 6) * 16
                               + lax.iota(jnp.int32, 16))
                        si[osl] = jnp.where(hit, offv, dum)
                        gi[osl] = sv * fp + p

                def fire_g(j):
                    gi = (gidx0, gidx1)[j % 2]
                    gr = (grows0, grows1)[j % 2]
                    return pltpu.async_copy(table.at[gi], gr, gsem)

                def fire_s(j):
                    si = (sidx0, sidx1)[j % 2]
                    gr = (grows0, grows1)[j % 2]
                    return pltpu.async_copy(gr, acc.at[si], ssem, add=True)

                compute_idx(0)
                gds = fire_g(0)
                sds = []
                for j in range(SB // B):
                    gds.wait()
                    sds.append(fire_s(j))
                    if j < SB // B - 1:
                        if j >= 1:
                            sds[j - 1].wait()
                        compute_idx(j + 1)
                        gds = fire_g(j + 1)
                sds[-2].wait()
                sds[-1].wait()
                return 0
            lax.fori_loop(0, nsb, sb_body, 0)
            plsc.subcore_barrier()

            for k in range(10):
                r0 = s * 6250 + k * DR
                pltpu.sync_copy(acc.at[pl.ds(r0, DR)], dbuf)
                pltpu.sync_copy(dbuf, out.at[p, c, pl.ds(r0, DR)])
            plsc.subcore_barrier()
            return 0
        lax.fori_loop(0, fp, pass_body, 0)

    return edge_pass


_edge_pass_1 = _make_edge_pass(1)
_edge_pass_4 = _make_edge_pass(4)


def _leaky_relu(x):
    return jnp.where(x >= 0, x, 0.1 * x)


def _layer_norm(x, g, b, eps=1e-5):
    m = jnp.mean(x, axis=-1, keepdims=True)
    v = jnp.mean((x - m) ** 2, axis=-1, keepdims=True)
    return (x - m) / jnp.sqrt(v + eps) * g + b


TB = 2000


def _layer1_body(x_ref, s1_ref, wc_ref, bp_ref, ws_ref, b_ref, g_ref,
                 bb_ref, h1_ref):
    agg = jnp.zeros((TB, HID), jnp.float32)
    for r in range(R):
        w_r = jnp.zeros((16, HID), jnp.float32)
        for bi in range(R):
            w_r = w_r + wc_ref[r, bi] * bp_ref[bi]
        agg = agg + jnp.dot(s1_ref[:, r * 16:(r + 1) * 16], w_r,
                            preferred_element_type=jnp.float32, precision=lax.Precision.HIGHEST)
    h = agg + jnp.dot(x_ref[...], ws_ref[...],
                      preferred_element_type=jnp.float32, precision=lax.Precision.HIGHEST) + b_ref[...]
    h = _leaky_relu(h)
    h1_ref[...] = _layer_norm(h, g_ref[...], bb_ref[...])


def _layer2_body(h1_ref, s2_ref, wc_ref, b2_ref, ws_ref, b_ref, g_ref,
                 bb_ref, gw1_ref, gb1_ref, gw2_ref, gb2_ref,
                 h2_ref, gate_ref):
    agg = jnp.zeros((TB, HID2), jnp.float32)
    for r in range(R):
        w_r = jnp.zeros((HID, HID2), jnp.float32)
        for bi in range(R):
            w_r = w_r + wc_ref[r, bi] * b2_ref[bi]
        agg = agg + jnp.dot(s2_ref[:, r * HID:(r + 1) * HID], w_r,
                            preferred_element_type=jnp.float32, precision=lax.Precision.HIGHEST)
    h = agg + jnp.dot(h1_ref[...], ws_ref[...],
                      preferred_element_type=jnp.float32, precision=lax.Precision.HIGHEST) + b_ref[...]
    h = _leaky_relu(h)
    h2 = _layer_norm(h, g_ref[...], bb_ref[...])
    h2_ref[...] = h2
    ga = jnp.maximum(jnp.dot(h2, gw1_ref[...],
                             preferred_element_type=jnp.float32, precision=lax.Precision.HIGHEST)
                     + gb1_ref[...], 0.0)
    gate_ref[...] = jnp.dot(ga, gw2_ref[...],
                            preferred_element_type=jnp.float32, precision=lax.Precision.HIGHEST) + gb2_ref[...]


def _pool_body(gate_ref, h2_ref, wmu_ref, bmu_ref, wlv_ref, blv_ref,
               mu_ref, lv_ref, acc_ref):
    i = pl.program_id(0)
    g = gate_ref[:, 0:1]
    m = jnp.max(g)
    ssum = jnp.sum(jnp.exp(g - m))
    gblk = gate_ref[pl.ds(i * TB, TB), :]
    w = jnp.exp(gblk - m) / ssum

    @pl.when(i == 0)
    def _():
        acc_ref[...] = jnp.zeros((8, HID2), jnp.float32)

    acc_ref[...] += lax.dot_general(
        w, h2_ref[...], (((0,), (0,)), ((), ())),
        preferred_element_type=jnp.float32, precision=lax.Precision.HIGHEST)
    ge = acc_ref[0:1, :]
    mu_ref[...] = jnp.clip(
        jnp.dot(ge, wmu_ref[...], preferred_element_type=jnp.float32, precision=lax.Precision.HIGHEST)
        + bmu_ref[...], -5.0, 5.0)
    lv_ref[...] = jnp.clip(
        jnp.dot(ge, wlv_ref[...], preferred_element_type=jnp.float32, precision=lax.Precision.HIGHEST)
        + blv_ref[...], -10.0, 10.0)


def kernel(node_feats, edge_index, edge_types, params):
    src = jnp.pad(edge_index[0], (0, EPAD))
    dst = jnp.pad(edge_index[1], (0, EPAD), constant_values=N)
    et = jnp.pad(edge_types, (0, EPAD))

    xpad = jnp.pad(node_feats, ((0, 0), (0, 16 - IN_DIM)))

    s1_raw = _edge_pass_1(xpad, src, dst, et)
    s1 = s1_raw.reshape(N, R * 16)

    bases1p = jnp.pad(params['bases1'], ((0, 0), (0, 16 - IN_DIM), (0, 0)))

    grid1 = N // TB
    h1 = pl.pallas_call(
        _layer1_body,
        grid=(grid1,),
        in_specs=[
            pl.BlockSpec((TB, IN_DIM), lambda i: (i, 0)),
            pl.BlockSpec((TB, R * 16), lambda i: (i, 0)),
            pl.BlockSpec((R, R), lambda i: (0, 0)),
            pl.BlockSpec((R, 16, HID), lambda i: (0, 0, 0)),
            pl.BlockSpec((IN_DIM, HID), lambda i: (0, 0)),
            pl.BlockSpec((HID,), lambda i: (0,)),
            pl.BlockSpec((HID,), lambda i: (0,)),
            pl.BlockSpec((HID,), lambda i: (0,)),
        ],
        out_specs=pl.BlockSpec((TB, HID), lambda i: (i, 0)),
        out_shape=jax.ShapeDtypeStruct((N, HID), jnp.float32),
    )(node_feats, s1, params['w_comp1'], bases1p, params['w_self1'],
      params['b1'], params['ln1_g'], params['ln1_b'])

    h1t = h1.reshape(N * R, 16)
    s2_raw = _edge_pass_4(h1t, src, dst, et)
    s2 = s2_raw.transpose(1, 2, 0, 3).reshape(N, R * HID)

    gw2_8 = jnp.tile(params['gate_w2'], (1, 8))
    gb2_8 = jnp.tile(params['gate_b2'], (8,))

    h2, gate8 = pl.pallas_call(
        _layer2_body,
        grid=(grid1,),
        in_specs=[
            pl.BlockSpec((TB, HID), lambda i: (i, 0)),
            pl.BlockSpec((TB, R * HID), lambda i: (i, 0)),
            pl.BlockSpec((R, R), lambda i: (0, 0)),
            pl.BlockSpec((R, HID, HID2), lambda i: (0, 0, 0)),
            pl.BlockSpec((HID, HID2), lambda i: (0, 0)),
            pl.BlockSpec((HID2,), lambda i: (0,)),
            pl.BlockSpec((HID2,), lambda i: (0,)),
            pl.BlockSpec((HID2,), lambda i: (0,)),
            pl.BlockSpec((HID2, HID), lambda i: (0, 0)),
            pl.BlockSpec((HID,), lambda i: (0,)),
            pl.BlockSpec((HID, 8), lambda i: (0, 0)),
            pl.BlockSpec((8,), lambda i: (0,)),
        ],
        out_specs=[
            pl.BlockSpec((TB, HID2), lambda i: (i, 0)),
            pl.BlockSpec((TB, 8), lambda i: (i, 0)),
        ],
        out_shape=[
            jax.ShapeDtypeStruct((N, HID2), jnp.float32),
            jax.ShapeDtypeStruct((N, 8), jnp.float32),
        ],
    )(h1, s2, params['w_comp2'], params['bases2'], params['w_self2'],
      params['b2'], params['ln2_g'], params['ln2_b'],
      params['gate_w1'], params['gate_b1'], gw2_8, gb2_8)

    mu, logvar = pl.pallas_call(
        _pool_body,
        grid=(grid1,),
        in_specs=[
            pl.BlockSpec((N, 8), lambda i: (0, 0)),
            pl.BlockSpec((TB, HID2), lambda i: (i, 0)),
            pl.BlockSpec((HID2, LATENT), lambda i: (0, 0)),
            pl.BlockSpec((LATENT,), lambda i: (0,)),
            pl.BlockSpec((HID2, LATENT), lambda i: (0, 0)),
            pl.BlockSpec((LATENT,), lambda i: (0,)),
        ],
        out_specs=[
            pl.BlockSpec((1, LATENT), lambda i: (0, 0)),
            pl.BlockSpec((1, LATENT), lambda i: (0, 0)),
        ],
        out_shape=[
            jax.ShapeDtypeStruct((1, LATENT), jnp.float32),
            jax.ShapeDtypeStruct((1, LATENT), jnp.float32),
        ],
        scratch_shapes=[pltpu.VMEM((8, HID2), jnp.float32)],
    )(gate8, h2, params['w_mu'], params['b_mu'],
      params['w_logvar'], params['b_logvar'])

    return (mu, logvar, h2)
